# baseline (speedup 1.0000x reference)
import functools

import jax
import jax.numpy as jnp
from jax import lax
from jax.experimental import pallas as pl
from jax.experimental.pallas import tpu as pltpu
from jax.experimental.pallas import tpu_sc as plsc

ACTION_N = 1_000_000

NUM_TILES = 32
SC_N = 65_536
SC_PER_TILE = SC_N // NUM_TILES
SC_UNROLL = 4
TC_N = ACTION_N - SC_N

LANES = 128
ROWS = 984
BLOCK = ROWS * LANES
GRID_TC = (TC_N + BLOCK - 1) // BLOCK

_ROTS_A = (13, 15, 26, 6)
_ROTS_B = (17, 29, 16, 24)


def _threefry_bernoulli(ctr, xv):
    ks0 = jnp.uint32(0)
    ks1 = jnp.uint32(1)
    ks2 = jnp.uint32(0x1BD11BDA) ^ ks0 ^ ks1
    ks = (ks0, ks1, ks2)

    x0 = jnp.zeros_like(ctr)
    x1 = ctr + ks1

    def rotl(v, r):
        return (v << jnp.uint32(r)) | (v >> jnp.uint32(32 - r))

    for g in range(5):
        rots = _ROTS_A if g % 2 == 0 else _ROTS_B
        for r in rots:
            x0 = x0 + x1
            x1 = rotl(x1, r)
            x1 = x1 ^ x0
        x0 = x0 + ks[(g + 1) % 3]
        x1 = x1 + ks[(g + 2) % 3] + jnp.uint32(g + 1)

    bits = x0 ^ x1
    fbits = (bits >> jnp.uint32(9)) | jnp.uint32(0x3F800000)
    rand = lax.bitcast_convert_type(fbits, jnp.float32) - jnp.float32(1.0)
    return jnp.where(rand < xv, jnp.float32(1.0), jnp.float32(0.0))


def _tc_block(x_ref, out_ref):
    pid = pl.program_id(0)
    base = (pid * BLOCK).astype(jnp.uint32)
    row = lax.broadcasted_iota(jnp.uint32, (ROWS, LANES), 0)
    lane = lax.broadcasted_iota(jnp.uint32, (ROWS, LANES), 1)
    ctr = base + row * jnp.uint32(LANES) + lane

    xv = x_ref[...].reshape(ROWS, LANES)
    out_ref[...] = _threefry_bernoulli(ctr, xv).reshape(BLOCK)


def _tc_head(x):
    return pl.pallas_call(
        _tc_block,
        out_shape=jax.ShapeDtypeStruct((TC_N,), jnp.float32),
        grid=(GRID_TC,),
        in_specs=[pl.BlockSpec((BLOCK,), lambda i: (i,))],
        out_specs=pl.BlockSpec((BLOCK,), lambda i: (i,)),
    )(x)


_sc_mesh = plsc.VectorSubcoreMesh(core_axis_name="c", subcore_axis_name="s")


@functools.partial(
    pl.kernel,
    mesh=_sc_mesh,
    out_type=jax.ShapeDtypeStruct((SC_N,), jnp.float32),
    scratch_types=[
        pltpu.VMEM((SC_PER_TILE,), jnp.float32),
        pltpu.VMEM((SC_PER_TILE,), jnp.float32),
    ],
)
def _sc_tail(x_hbm, out_hbm, x_v, out_v):
    wid = lax.axis_index("s") * 2 + lax.axis_index("c")
    base = wid * SC_PER_TILE
    pltpu.sync_copy(x_hbm.at[pl.ds(TC_N + base, SC_PER_TILE)], x_v)

    lane = lax.iota(jnp.uint32, 16)
    gbase = jnp.uint32(TC_N) + base.astype(jnp.uint32)

    def body(j, carry):
        off0 = j * (16 * SC_UNROLL)
        for u in range(SC_UNROLL):
            off = off0 + u * 16
            ctr = gbase + off.astype(jnp.uint32) + lane
            xv = x_v[pl.ds(off, 16)]
            out_v[pl.ds(off, 16)] = _threefry_bernoulli(ctr, xv)
        return carry

    lax.fori_loop(0, SC_PER_TILE // (16 * SC_UNROLL), body, 0)
    pltpu.sync_copy(out_v, out_hbm.at[pl.ds(base, SC_PER_TILE)])


def kernel(x):
    sc_out = _sc_tail(x)
    tc_out = _tc_head(x)
    return jnp.concatenate([tc_out, sc_out])


# speedup vs baseline: 2.0213x; 2.0213x over previous
import numpy as np
import jax
import jax.numpy as jnp
from jax import lax
from jax.experimental import pallas as pl

ACTION_N = 1_000_000
LANES = 128
ROWS = 3912
BLOCK = ROWS * LANES
GRID = 2

_CTR1 = np.arange(1, GRID * BLOCK + 1, dtype=np.uint32)

_ROTS_A = (13, 15, 26, 6)
_ROTS_B = (17, 29, 16, 24)


def _threefry_bernoulli(x1, xv):
    ks0 = jnp.uint32(0)
    ks1 = jnp.uint32(1)
    ks2 = jnp.uint32(0x1BD11BDA) ^ ks0 ^ ks1
    ks = (ks0, ks1, ks2)

    x0 = jnp.zeros_like(x1)

    def rotl(v, r):
        return (v << jnp.uint32(r)) | (v >> jnp.uint32(32 - r))

    for g in range(5):
        rots = _ROTS_A if g % 2 == 0 else _ROTS_B
        for r in rots:
            x0 = x0 + x1
            x1 = rotl(x1, r)
            x1 = x1 ^ x0
        x0 = x0 + ks[(g + 1) % 3]
        x1 = x1 + ks[(g + 2) % 3] + jnp.uint32(g + 1)

    bits = x0 ^ x1
    fbits = (bits >> jnp.uint32(9)) | jnp.uint32(0x3F800000)
    rand = lax.bitcast_convert_type(fbits, jnp.float32) - jnp.float32(1.0)
    return jnp.where(rand < xv, jnp.float32(1.0), jnp.float32(0.0))


def _bernoulli_block(x_ref, c_ref, out_ref):
    xv = x_ref[...].reshape(ROWS, LANES)
    x1 = c_ref[...].reshape(ROWS, LANES)
    out_ref[...] = _threefry_bernoulli(x1, xv).reshape(BLOCK)


def kernel(x):
    return pl.pallas_call(
        _bernoulli_block,
        out_shape=jax.ShapeDtypeStruct((ACTION_N,), jnp.float32),
        grid=(GRID,),
        in_specs=[
            pl.BlockSpec((BLOCK,), lambda i: (i,)),
            pl.BlockSpec((BLOCK,), lambda i: (i,)),
        ],
        out_specs=pl.BlockSpec((BLOCK,), lambda i: (i,)),
    )(x, jnp.asarray(_CTR1))


# speedup vs baseline: 2.1241x; 1.0509x over previous
import numpy as np
import jax
import jax.numpy as jnp
from jax import lax
from jax.experimental import pallas as pl

ACTION_N = 1_000_000
LANES = 128
ROWS = 984
BLOCK = ROWS * LANES
GRID = 8

_CTR1 = np.arange(1, GRID * BLOCK + 1, dtype=np.uint32)

_ROTS_A = (13, 15, 26, 6)
_ROTS_B = (17, 29, 16, 24)


def _threefry_bernoulli(x1, xv):
    ks0 = jnp.uint32(0)
    ks1 = jnp.uint32(1)
    ks2 = jnp.uint32(0x1BD11BDA) ^ ks0 ^ ks1
    ks = (ks0, ks1, ks2)

    x0 = jnp.zeros_like(x1)

    def rotl(v, r):
        return (v << jnp.uint32(r)) | (v >> jnp.uint32(32 - r))

    for g in range(5):
        rots = _ROTS_A if g % 2 == 0 else _ROTS_B
        for r in rots:
            x0 = x0 + x1
            x1 = rotl(x1, r)
            x1 = x1 ^ x0
        x0 = x0 + ks[(g + 1) % 3]
        x1 = x1 + ks[(g + 2) % 3] + jnp.uint32(g + 1)

    bits = x0 ^ x1
    fbits = (bits >> jnp.uint32(9)) | jnp.uint32(0x3F800000)
    rand = lax.bitcast_convert_type(fbits, jnp.float32) - jnp.float32(1.0)
    return jnp.where(rand < xv, jnp.float32(1.0), jnp.float32(0.0))


def _bernoulli_block(x_ref, c_ref, out_ref):
    xv = x_ref[...].reshape(ROWS, LANES)
    x1 = c_ref[...].reshape(ROWS, LANES)
    out_ref[...] = _threefry_bernoulli(x1, xv).reshape(BLOCK)


def kernel(x):
    return pl.pallas_call(
        _bernoulli_block,
        out_shape=jax.ShapeDtypeStruct((ACTION_N,), jnp.float32),
        grid=(GRID,),
        in_specs=[
            pl.BlockSpec((BLOCK,), lambda i: (i,)),
            pl.BlockSpec((BLOCK,), lambda i: (i,)),
        ],
        out_specs=pl.BlockSpec((BLOCK,), lambda i: (i,)),
    )(x, jnp.asarray(_CTR1))


# speedup vs baseline: 2.1432x; 1.0090x over previous
import numpy as np
import jax
import jax.numpy as jnp
from jax import lax
from jax.experimental import pallas as pl

ACTION_N = 1_000_000
LANES = 128
ROWS = 784
BLOCK = ROWS * LANES
GRID = 10

_CTR1 = np.arange(1, GRID * BLOCK + 1, dtype=np.uint32)

_ROTS_A = (13, 15, 26, 6)
_ROTS_B = (17, 29, 16, 24)


def _threefry_bernoulli(x1, xv):
    ks0 = jnp.uint32(0)
    ks1 = jnp.uint32(1)
    ks2 = jnp.uint32(0x1BD11BDA) ^ ks0 ^ ks1
    ks = (ks0, ks1, ks2)

    x0 = jnp.zeros_like(x1)

    def rotl(v, r):
        return (v << jnp.uint32(r)) | (v >> jnp.uint32(32 - r))

    for g in range(5):
        rots = _ROTS_A if g % 2 == 0 else _ROTS_B
        for r in rots:
            x0 = x0 + x1
            x1 = rotl(x1, r)
            x1 = x1 ^ x0
        x0 = x0 + ks[(g + 1) % 3]
        x1 = x1 + ks[(g + 2) % 3] + jnp.uint32(g + 1)

    bits = x0 ^ x1
    fbits = (bits >> jnp.uint32(9)) | jnp.uint32(0x3F800000)
    rand = lax.bitcast_convert_type(fbits, jnp.float32) - jnp.float32(1.0)
    return jnp.where(rand < xv, jnp.float32(1.0), jnp.float32(0.0))


def _bernoulli_block(x_ref, c_ref, out_ref):
    xv = x_ref[...].reshape(ROWS, LANES)
    x1 = c_ref[...].reshape(ROWS, LANES)
    out_ref[...] = _threefry_bernoulli(x1, xv).reshape(BLOCK)


def kernel(x):
    return pl.pallas_call(
        _bernoulli_block,
        out_shape=jax.ShapeDtypeStruct((ACTION_N,), jnp.float32),
        grid=(GRID,),
        in_specs=[
            pl.BlockSpec((BLOCK,), lambda i: (i,)),
            pl.BlockSpec((BLOCK,), lambda i: (i,)),
        ],
        out_specs=pl.BlockSpec((BLOCK,), lambda i: (i,)),
    )(x, jnp.asarray(_CTR1))
